# baseline (device time: 102743 ns/iter reference)
import jax
import jax.numpy as jnp
from jax import lax
from jax.experimental import pallas as pl
from jax.experimental.pallas import tpu as pltpu

N_DEV = 8
B, Sq, Hq, Dh = 2, 128, 4, 64
SKV_SH = 128
SKV = N_DEV * SKV_SH
D_MODEL = 512
BLK = 64


def kernel(x, Wq, K_ext, V_ext, Wo):
    K_t = jnp.transpose(K_ext, (0, 2, 1, 3))
    V_t = jnp.transpose(V_ext, (0, 2, 1, 3))

    def body(x_ref, wq_ref, k_ref, v_ref, wo_ref, out_ref,
             k_all, v_all, send_sems, recv_sems):
        my = lax.axis_index("i")
        left = lax.rem(my + N_DEV - 1, N_DEV)
        right = lax.rem(my + 1, N_DEV)

        barrier_sem = pltpu.get_barrier_semaphore()
        pl.semaphore_signal(barrier_sem, inc=1, device_id=(left,),
                            device_id_type=pl.DeviceIdType.MESH)
        pl.semaphore_signal(barrier_sem, inc=1, device_id=(right,),
                            device_id_type=pl.DeviceIdType.MESH)
        pl.semaphore_wait(barrier_sem, 2)

        k_all[my] = k_ref[...]
        v_all[my] = v_ref[...]

        for h in range(N_DEV - 1):
            src = lax.rem(my - h + N_DEV, N_DEV)
            rk = pltpu.make_async_remote_copy(
                src_ref=k_all.at[src], dst_ref=k_all.at[src],
                send_sem=send_sems.at[2 * h], recv_sem=recv_sems.at[2 * h],
                device_id=(right,), device_id_type=pl.DeviceIdType.MESH)
            rv = pltpu.make_async_remote_copy(
                src_ref=v_all.at[src], dst_ref=v_all.at[src],
                send_sem=send_sems.at[2 * h + 1],
                recv_sem=recv_sems.at[2 * h + 1],
                device_id=(right,), device_id_type=pl.DeviceIdType.MESH)
            rk.start()
            rv.start()
            rk.wait()
            rv.wait()

        x2 = x_ref[...].reshape(B * Sq, D_MODEL)
        q = jnp.dot(x2, wq_ref[...], preferred_element_type=jnp.float32)

        row = lax.broadcasted_iota(jnp.int32, (Sq, SKV), 0) // BLK
        col = lax.broadcasted_iota(jnp.int32, (Sq, SKV), 1) // BLK
        mask = (row == col) | (col == 0) | (lax.rem(row + col, 3) == 0)

        ctx_rows = []
        for b in range(B):
            head_cols = []
            for hq in range(Hq):
                qbh = q[b * Sq:(b + 1) * Sq, hq * Dh:(hq + 1) * Dh]
                kbh = jnp.concatenate(
                    [k_all[c, b, hq] for c in range(N_DEV)], axis=0)
                vbh = jnp.concatenate(
                    [v_all[c, b, hq] for c in range(N_DEV)], axis=0)
                s = lax.dot_general(
                    qbh, kbh, (((1,), (1,)), ((), ())),
                    preferred_element_type=jnp.float32) * 0.125
                s = jnp.where(mask, s, -1e9)
                m = jnp.max(s, axis=-1, keepdims=True)
                w = jnp.exp(s - m)
                w = w / jnp.sum(w, axis=-1, keepdims=True)
                head_cols.append(
                    jnp.dot(w, vbh, preferred_element_type=jnp.float32))
            ctx_rows.append(jnp.concatenate(head_cols, axis=1))
        ctx = jnp.concatenate(ctx_rows, axis=0)
        out = jnp.dot(ctx, wo_ref[...], preferred_element_type=jnp.float32)
        out_ref[...] = out.reshape(B, Sq, D_MODEL)

    return pl.pallas_call(
        body,
        out_shape=jax.ShapeDtypeStruct((B, Sq, D_MODEL), jnp.float32),
        in_specs=[pl.BlockSpec(memory_space=pltpu.VMEM)] * 5,
        out_specs=pl.BlockSpec(memory_space=pltpu.VMEM),
        scratch_shapes=[
            pltpu.VMEM((N_DEV, B, Hq, SKV_SH, Dh), jnp.float32),
            pltpu.VMEM((N_DEV, B, Hq, SKV_SH, Dh), jnp.float32),
            pltpu.SemaphoreType.DMA((2 * (N_DEV - 1),)),
            pltpu.SemaphoreType.DMA((2 * (N_DEV - 1),)),
        ],
        compiler_params=pltpu.CompilerParams(collective_id=0),
    )(x, Wq, K_t, V_t, Wo)


# device time: 53832 ns/iter; 1.9086x vs baseline; 1.9086x over previous
import jax
import jax.numpy as jnp
from jax import lax
from jax.experimental import pallas as pl
from jax.experimental.pallas import tpu as pltpu

N_DEV = 8
B, Sq, Hq, Dh = 2, 128, 4, 64
SKV_SH = 128
D_MODEL = 512
BLK = 64
BH = B * Hq

ROUNDS = ((1, 7, 1), (3, 6, 2), (4, 4, 4))


def kernel(x, Wq, K_ext, V_ext, Wo):
    K_t = jnp.transpose(K_ext, (0, 2, 1, 3))
    V_t = jnp.transpose(V_ext, (0, 2, 1, 3))

    def body(x_ref, wq_ref, k_ref, v_ref, wo_ref, out_ref,
             ctx_all, l_all, send_sems, recv_sems):
        my = lax.axis_index("i")

        x2 = x_ref[...].reshape(B * Sq, D_MODEL)
        q = jnp.dot(x2, wq_ref[...], preferred_element_type=jnp.float32)

        rowb = lax.broadcasted_iota(jnp.int32, (Sq, SKV_SH), 0) // BLK
        colb = 2 * my + lax.broadcasted_iota(jnp.int32, (Sq, SKV_SH), 1) // BLK
        mask = (rowb == colb) | (colb == 0) | (lax.rem(rowb + colb, 3) == 0)

        for b in range(B):
            for h in range(Hq):
                qbh = q[b * Sq:(b + 1) * Sq, h * Dh:(h + 1) * Dh]
                s = lax.dot_general(
                    qbh, k_ref[b, h], (((1,), (1,)), ((), ())),
                    preferred_element_type=jnp.float32) * 0.125
                p = jnp.where(mask, jnp.exp(s), 0.0)
                ctx_all[my, b, h] = jnp.dot(
                    p, v_ref[b, h], preferred_element_type=jnp.float32)
                l_all[my, b * Hq + h, :] = jnp.sum(p, axis=-1)

        barrier_sem = pltpu.get_barrier_semaphore()
        for xr, _, _ in ROUNDS:
            pl.semaphore_signal(
                barrier_sem, inc=1,
                device_id=(jnp.bitwise_xor(my, xr),),
                device_id_type=pl.DeviceIdType.MESH)
        pl.semaphore_wait(barrier_sem, 3)

        for r, (xr, blkmask, cnt) in enumerate(ROUNDS):
            partner = jnp.bitwise_xor(my, xr)
            base = jnp.bitwise_and(my, blkmask)
            rc = pltpu.make_async_remote_copy(
                src_ref=ctx_all.at[pl.ds(base, cnt)],
                dst_ref=ctx_all.at[pl.ds(base, cnt)],
                send_sem=send_sems.at[2 * r], recv_sem=recv_sems.at[2 * r],
                device_id=(partner,), device_id_type=pl.DeviceIdType.MESH)
            rl = pltpu.make_async_remote_copy(
                src_ref=l_all.at[pl.ds(base, cnt)],
                dst_ref=l_all.at[pl.ds(base, cnt)],
                send_sem=send_sems.at[2 * r + 1],
                recv_sem=recv_sems.at[2 * r + 1],
                device_id=(partner,), device_id_type=pl.DeviceIdType.MESH)
            rc.start()
            rl.start()
            rc.wait()
            rl.wait()

        ctx_sum = jnp.sum(ctx_all[...], axis=0)
        l_sum = jnp.sum(l_all[...], axis=0)
        l_t = jnp.transpose(l_sum)

        ctx_rows = []
        for b in range(B):
            head_cols = []
            for h in range(Hq):
                bh = b * Hq + h
                head_cols.append(ctx_sum[b, h] / l_t[:, bh:bh + 1])
            ctx_rows.append(jnp.concatenate(head_cols, axis=1))
        ctx = jnp.concatenate(ctx_rows, axis=0)
        out = jnp.dot(ctx, wo_ref[...], preferred_element_type=jnp.float32)
        out_ref[...] = out.reshape(B, Sq, D_MODEL)

    return pl.pallas_call(
        body,
        out_shape=jax.ShapeDtypeStruct((B, Sq, D_MODEL), jnp.float32),
        in_specs=[pl.BlockSpec(memory_space=pltpu.VMEM)] * 5,
        out_specs=pl.BlockSpec(memory_space=pltpu.VMEM),
        scratch_shapes=[
            pltpu.VMEM((N_DEV, B, Hq, Sq, Dh), jnp.float32),
            pltpu.VMEM((N_DEV, BH, Sq), jnp.float32),
            pltpu.SemaphoreType.DMA((6,)),
            pltpu.SemaphoreType.DMA((6,)),
        ],
        compiler_params=pltpu.CompilerParams(collective_id=0),
    )(x, Wq, K_t, V_t, Wo)
